# baseline (device time: 20168 ns/iter reference)
import jax
import jax.numpy as jnp
from jax import lax
from jax.experimental import pallas as pl
from jax.experimental.pallas import tpu as pltpu

N_DEV = 16


def kernel(x, w_mat):
    k, n_per = x.shape
    _, n = w_mat.shape
    m_per = k // N_DEV

    def body(x_ref, w_ref, out_ref, slab_ref, send_sems, recv_sems, local_sem):
        my = lax.axis_index("i")

        local = pltpu.make_async_copy(
            x_ref.at[pl.ds(my * m_per, m_per), :],
            slab_ref.at[my],
            local_sem,
        )
        local.start()

        sends = []
        for d in range(1, N_DEV):
            t = lax.rem(my + d, N_DEV)
            rdma = pltpu.make_async_remote_copy(
                src_ref=x_ref.at[pl.ds(t * m_per, m_per), :],
                dst_ref=slab_ref.at[my],
                send_sem=send_sems.at[d - 1],
                recv_sem=recv_sems.at[my],
                device_id=(t,),
                device_id_type=pl.DeviceIdType.MESH,
            )
            rdma.start()
            sends.append(rdma)

        local.wait()

        for d in range(1, N_DEV):
            s = lax.rem(my + d, N_DEV)
            recv = pltpu.make_async_remote_copy(
                src_ref=x_ref.at[pl.ds(s * m_per, m_per), :],
                dst_ref=slab_ref.at[s],
                send_sem=send_sems.at[d - 1],
                recv_sem=recv_sems.at[s],
                device_id=(s,),
                device_id_type=pl.DeviceIdType.MESH,
            )
            recv.wait_recv()

        xrow = jnp.concatenate(
            [slab_ref[j] for j in range(N_DEV)], axis=1
        )
        out_ref[:, :] = jnp.dot(
            xrow, w_ref[:, :], preferred_element_type=jnp.float32
        )

        for rdma in sends:
            rdma.wait_send()

    return pl.pallas_call(
        body,
        out_shape=jax.ShapeDtypeStruct((m_per, n), jnp.float32),
        in_specs=[
            pl.BlockSpec(memory_space=pltpu.VMEM),
            pl.BlockSpec(memory_space=pltpu.VMEM),
        ],
        out_specs=pl.BlockSpec(memory_space=pltpu.VMEM),
        scratch_shapes=[
            pltpu.VMEM((N_DEV, m_per, n_per), jnp.float32),
            pltpu.SemaphoreType.DMA((N_DEV - 1,)),
            pltpu.SemaphoreType.DMA((N_DEV,)),
            pltpu.SemaphoreType.DMA,
        ],
    )(x, w_mat)


# device time: 16013 ns/iter; 1.2595x vs baseline; 1.2595x over previous
import jax
import jax.numpy as jnp
from jax import lax
from jax.experimental import pallas as pl
from jax.experimental.pallas import tpu as pltpu

N_DEV = 16


def kernel(x, w_mat):
    k, n_per = x.shape
    _, n = w_mat.shape
    m_per = k // N_DEV

    def body(x_ref, w_ref, out_ref, slab_ref, send_sems, recv_sems, local_sem):
        my = lax.axis_index("i")

        with jax.named_scope("a2a_barrier"):
            barrier_sem = pltpu.get_barrier_semaphore()
            for d in range(1, N_DEV):
                t = lax.rem(my + d, N_DEV)
                pl.semaphore_signal(
                    barrier_sem, inc=1,
                    device_id=(t,), device_id_type=pl.DeviceIdType.MESH,
                )
            pl.semaphore_wait(barrier_sem, N_DEV - 1)

        with jax.named_scope("a2a_send"):
            local = pltpu.make_async_copy(
                x_ref.at[pl.ds(my * m_per, m_per), :],
                slab_ref.at[my],
                local_sem,
            )
            local.start()

            sends = []
            for d in range(1, N_DEV):
                t = lax.rem(my + d, N_DEV)
                rdma = pltpu.make_async_remote_copy(
                    src_ref=x_ref.at[pl.ds(t * m_per, m_per), :],
                    dst_ref=slab_ref.at[my],
                    send_sem=send_sems.at[d - 1],
                    recv_sem=recv_sems.at[my],
                    device_id=(t,),
                    device_id_type=pl.DeviceIdType.MESH,
                )
                rdma.start()
                sends.append(rdma)
            local.wait()

        with jax.named_scope("a2a_wait_recv"):
            for d in range(1, N_DEV):
                s = lax.rem(my + d, N_DEV)
                recv = pltpu.make_async_remote_copy(
                    src_ref=x_ref.at[pl.ds(s * m_per, m_per), :],
                    dst_ref=slab_ref.at[s],
                    send_sem=send_sems.at[d - 1],
                    recv_sem=recv_sems.at[s],
                    device_id=(s,),
                    device_id_type=pl.DeviceIdType.MESH,
                )
                recv.wait_recv()

        with jax.named_scope("gemm"):
            xrow = jnp.concatenate(
                [slab_ref[j] for j in range(N_DEV)], axis=1
            )
            out_ref[:, :] = jnp.dot(
                xrow, w_ref[:, :], preferred_element_type=jnp.float32
            )

        with jax.named_scope("drain_sends"):
            for rdma in sends:
                rdma.wait_send()

    return pl.pallas_call(
        body,
        out_shape=jax.ShapeDtypeStruct((m_per, n), jnp.float32),
        in_specs=[
            pl.BlockSpec(memory_space=pltpu.VMEM),
            pl.BlockSpec(memory_space=pltpu.VMEM),
        ],
        out_specs=pl.BlockSpec(memory_space=pltpu.VMEM),
        scratch_shapes=[
            pltpu.VMEM((N_DEV, m_per, n_per), jnp.float32),
            pltpu.SemaphoreType.DMA((N_DEV - 1,)),
            pltpu.SemaphoreType.DMA((N_DEV,)),
            pltpu.SemaphoreType.DMA,
        ],
        compiler_params=pltpu.CompilerParams(collective_id=0),
    )(x, w_mat)


# device time: 15995 ns/iter; 1.2609x vs baseline; 1.0011x over previous
import jax
import jax.numpy as jnp
from jax import lax
from jax.experimental import pallas as pl
from jax.experimental.pallas import tpu as pltpu

N_DEV = 16
GROUP = 4


def kernel(x, w_mat):
    k, n_per = x.shape
    _, n = w_mat.shape
    m_per = k // N_DEV

    def body(x_ref, w_ref, out_ref, slab_ref, send_sems, recv_sems,
             credit_sems, local_sem):
        my = lax.axis_index("i")

        bsem = pltpu.get_barrier_semaphore()
        pl.semaphore_signal(bsem, 1)
        pl.semaphore_wait(bsem, 1)

        for d in range(1, N_DEV):
            t = lax.rem(my + d, N_DEV)
            pl.semaphore_signal(
                credit_sems.at[my], inc=1,
                device_id=(t,), device_id_type=pl.DeviceIdType.MESH,
            )

        local = pltpu.make_async_copy(
            x_ref.at[pl.ds(my * m_per, m_per), :],
            slab_ref.at[0],
            local_sem,
        )
        local.start()

        sends = []
        for d in range(1, N_DEV):
            t = lax.rem(my + d, N_DEV)
            pl.semaphore_wait(credit_sems.at[t], 1)
            rdma = pltpu.make_async_remote_copy(
                src_ref=x_ref.at[pl.ds(t * m_per, m_per), :],
                dst_ref=slab_ref.at[d],
                send_sem=send_sems.at[d - 1],
                recv_sem=recv_sems.at[d],
                device_id=(t,),
                device_id_type=pl.DeviceIdType.MESH,
            )
            rdma.start()
            sends.append(rdma)

        local.wait()

        for g in range(N_DEV // GROUP):
            for d in range(g * GROUP, (g + 1) * GROUP):
                if d == 0:
                    continue
                recv = pltpu.make_async_remote_copy(
                    src_ref=x_ref.at[pl.ds(0, m_per), :],
                    dst_ref=slab_ref.at[d],
                    send_sem=send_sems.at[d - 1],
                    recv_sem=recv_sems.at[d],
                    device_id=(my,),
                    device_id_type=pl.DeviceIdType.MESH,
                )
                recv.wait_recv()
            xg = jnp.concatenate(
                [slab_ref[d] for d in range(g * GROUP, (g + 1) * GROUP)],
                axis=1,
            )
            wg = jnp.concatenate(
                [
                    w_ref[pl.ds(lax.rem(my - d + N_DEV, N_DEV) * m_per,
                                m_per), :]
                    for d in range(g * GROUP, (g + 1) * GROUP)
                ],
                axis=0,
            )
            part = jnp.dot(xg, wg, preferred_element_type=jnp.float32)
            if g == 0:
                out_ref[:, :] = part
            else:
                out_ref[:, :] += part

        for rdma in sends:
            rdma.wait_send()

    return pl.pallas_call(
        body,
        out_shape=jax.ShapeDtypeStruct((m_per, n), jnp.float32),
        in_specs=[
            pl.BlockSpec(memory_space=pltpu.VMEM),
            pl.BlockSpec(memory_space=pltpu.VMEM),
        ],
        out_specs=pl.BlockSpec(memory_space=pltpu.VMEM),
        scratch_shapes=[
            pltpu.VMEM((N_DEV, m_per, n_per), jnp.float32),
            pltpu.SemaphoreType.DMA((N_DEV - 1,)),
            pltpu.SemaphoreType.DMA((N_DEV,)),
            pltpu.SemaphoreType.REGULAR((N_DEV,)),
            pltpu.SemaphoreType.DMA,
        ],
        compiler_params=pltpu.CompilerParams(collective_id=0),
    )(x, w_mat)
